# baseline (speedup 1.0000x reference)
import functools

import jax
import jax.numpy as jnp
from jax import lax
from jax.experimental import pallas as pl
from jax.experimental.pallas import tpu as pltpu
from jax.experimental.pallas import tpu_sc as plsc

_B = 4096
_D = 128
_R = 128


def _make_lookup():
    info = plsc.get_sparse_core_info()
    nc = info.num_cores
    rows_per_c = _B // nc
    mesh = plsc.ScalarSubcoreMesh(axis_name="c", num_cores=nc)

    @functools.partial(
        pl.kernel,
        mesh=mesh,
        out_type=jax.ShapeDtypeStruct((_B, _D), jnp.float32),
        scratch_types=[
            pltpu.VMEM_SHARED((_R, _D), jnp.float32),
            pltpu.SemaphoreType.DMA,
        ],
    )
    def lookup(idx_hbm, table_hbm, out_hbm, spbuf, sem):
        cid = lax.axis_index("c")
        base = cid * rows_per_c
        pltpu.sync_copy(table_hbm, spbuf.at[pl.ds(0, 1)])
        n = 1
        while n < _R:
            pltpu.sync_copy(spbuf.at[pl.ds(0, n)], spbuf.at[pl.ds(n, n)])
            n *= 2
        copies = [
            pltpu.async_copy(spbuf, out_hbm.at[pl.ds(base + k * _R, _R)], sem)
            for k in range(rows_per_c // _R)
        ]
        for c in copies:
            c.wait()

    return lookup


_lookup = _make_lookup()


@jax.jit
def kernel(x, table):
    return _lookup(x.astype(jnp.int32), table)


# speedup vs baseline: 7.2719x; 7.2719x over previous
import jax
import jax.numpy as jnp
from jax.experimental import pallas as pl

_B = 4096
_D = 128
_G = 8
_BLK = _B // _G


def _bcast(table_ref, out_ref):
    out_ref[...] = jnp.broadcast_to(table_ref[...], (_BLK, _D))


@jax.jit
def kernel(x, table):
    del x
    return pl.pallas_call(
        _bcast,
        grid=(_G,),
        in_specs=[pl.BlockSpec((1, _D), lambda i: (0, 0))],
        out_specs=pl.BlockSpec((_BLK, _D), lambda i: (i, 0)),
        out_shape=jax.ShapeDtypeStruct((_B, _D), jnp.float32),
    )(table)


# speedup vs baseline: 13.6606x; 1.8786x over previous
import jax
import jax.numpy as jnp
from jax.experimental import pallas as pl
from jax.experimental.pallas import tpu as pltpu

_B = 4096
_D = 128
_TILE = 512
_NDMA = _B // _TILE


def _bcast(table_ref, out_hbm, tile_v, sem):
    tile_v[...] = jnp.broadcast_to(table_ref[...], (_TILE, _D))
    copies = [
        pltpu.make_async_copy(tile_v, out_hbm.at[pl.ds(k * _TILE, _TILE)], sem)
        for k in range(_NDMA)
    ]
    for c in copies:
        c.start()
    for c in copies:
        c.wait()


@jax.jit
def kernel(x, table):
    del x
    return pl.pallas_call(
        _bcast,
        out_specs=pl.BlockSpec(memory_space=pl.ANY),
        out_shape=jax.ShapeDtypeStruct((_B, _D), jnp.float32),
        scratch_shapes=[
            pltpu.VMEM((_TILE, _D), jnp.float32),
            pltpu.SemaphoreType.DMA,
        ],
    )(table)
